# Initial kernel scaffold; baseline (speedup 1.0000x reference)
import jax
import jax.numpy as jnp
from jax.experimental import pallas as pl


def kernel(points, features, W0, W1, W2, Wsc, g0, b0, g1, b1, g2, b2, gsc, bsc):
    raise NotImplementedError("write your pallas kernel here")


# speedup vs baseline: 11.9738x; 11.9738x over previous
import functools

import jax
import jax.numpy as jnp
from jax import lax
from jax.experimental import pallas as pl
from jax.experimental.pallas import tpu as pltpu
from jax.experimental.pallas import tpu_sc as plsc

_B, _N, _K, _DP, _C = 8, 2048, 16, 3, 64
_EPS = 1e-3
_RT = 512
_R2 = 512
_E = _B * _N * _K
_NC, _NS = 2, 16
_NW = _NC * _NS
_EPW = _E // _NW
_CHUNK = 128
_NCHUNK = _EPW // _CHUNK


def _topk_body(pts_ref, ptsT_ref, idx_ref):
    b = pl.program_id(0)
    p_tile = pts_ref[0]
    p_all_t = ptsT_ref[0]
    r_tile = jnp.sum(p_tile * p_tile, axis=1, keepdims=True)
    r_all = jnp.sum(p_all_t * p_all_t, axis=0, keepdims=True)
    m = jnp.dot(p_tile, p_all_t, preferred_element_type=jnp.float32)
    work = r_tile - 2.0 * m + r_all
    iota = lax.broadcasted_iota(jnp.int32, (_RT, _N), 1)
    cols = []
    for k in range(_K + 1):
        mval = jnp.min(work, axis=1, keepdims=True)
        cand = jnp.where(work == mval, iota, _N)
        midx = jnp.min(cand, axis=1, keepdims=True)
        if k > 0:
            cols.append(midx)
        work = jnp.where(iota == midx, jnp.float32(jnp.inf), work)
    idx_ref[0] = jnp.concatenate(cols, axis=1) + b * _N


_topk_call = pl.pallas_call(
    _topk_body,
    grid=(_B, _N // _RT),
    in_specs=[
        pl.BlockSpec((1, _RT, _DP), lambda b, i: (b, i, 0)),
        pl.BlockSpec((1, _DP, _N), lambda b, i: (b, 0, 0)),
    ],
    out_specs=pl.BlockSpec((1, _RT, _K + 1 - 1), lambda b, i: (b, i, 0)),
    out_shape=jax.ShapeDtypeStruct((_B, _N, _K), jnp.int32),
)


def _gather_body(feat_hbm, idx_hbm, out_hbm, idx_v, rows_v, sem):
    wid = lax.axis_index("s") * _NC + lax.axis_index("c")
    base = wid * _EPW
    pltpu.sync_copy(idx_hbm.at[pl.ds(wid * _NCHUNK, _NCHUNK)], idx_v)

    def chunk(j, carry):
        pltpu.async_copy(feat_hbm.at[idx_v.at[j]], rows_v, sem).wait()
        pltpu.sync_copy(rows_v, out_hbm.at[pl.ds(base + j * _CHUNK, _CHUNK)])
        return carry

    lax.fori_loop(0, _NCHUNK, chunk, 0)


@functools.lru_cache(maxsize=1)
def _gather_call():
    return pl.kernel(
        _gather_body,
        out_type=jax.ShapeDtypeStruct((_E, _C), jnp.float32),
        mesh=plsc.VectorSubcoreMesh(core_axis_name="c", subcore_axis_name="s"),
        scratch_types=[
            pltpu.VMEM((_NCHUNK, _CHUNK), jnp.int32),
            pltpu.VMEM((_CHUNK, _C), jnp.float32),
            pltpu.SemaphoreType.DMA,
        ],
        compiler_params=pltpu.CompilerParams(use_tc_tiling_on_sc=False),
    )


def _mlp_body(knn_ref, f_ref, w0a_ref, w0b_ref, w1_ref, w2_ref, wsc_ref,
              b0_ref, b1_ref, b2_ref, bsc_ref, o_ref):
    f = f_ref[...]
    d = knn_ref[...].reshape(_R2, _K, _C) - f[:, None, :]
    y0 = jnp.dot(f, w0a_ref[...], preferred_element_type=jnp.float32)
    z0 = jnp.dot(d.reshape(_R2 * _K, _C), w0b_ref[...],
                 preferred_element_type=jnp.float32)
    a0 = jnp.maximum(z0.reshape(_R2, _K, _C) + y0[:, None, :] + b0_ref[...],
                     0.0)
    a1 = jnp.maximum(
        jnp.dot(a0.reshape(_R2 * _K, _C), w1_ref[...],
                preferred_element_type=jnp.float32) + b1_ref[...], 0.0)
    a2 = jnp.maximum(
        jnp.dot(a1, w2_ref[...], preferred_element_type=jnp.float32)
        + b2_ref[...], 0.0)
    fts = jnp.sum(a2.reshape(_R2, _K, _C), axis=1) * (1.0 / _K)
    sc = jnp.dot(f, wsc_ref[...], preferred_element_type=jnp.float32) \
        + bsc_ref[...]
    o_ref[...] = jnp.maximum(sc + fts, 0.0)


_mlp_call = pl.pallas_call(
    _mlp_body,
    grid=(_B * _N // _R2,),
    in_specs=[
        pl.BlockSpec((_R2 * _K, _C), lambda i: (i, 0)),
        pl.BlockSpec((_R2, _C), lambda i: (i, 0)),
    ] + [pl.BlockSpec((_C, _C), lambda i: (0, 0))] * 5
      + [pl.BlockSpec((1, _C), lambda i: (0, 0))] * 4,
    out_specs=pl.BlockSpec((_R2, _C), lambda i: (i, 0)),
    out_shape=jax.ShapeDtypeStruct((_B * _N, _C), jnp.float32),
)


def kernel(points, features, W0, W1, W2, Wsc,
           g0, b0, g1, b1, g2, b2, gsc, bsc):
    s = 1.0 / jnp.sqrt(jnp.float32(1.0 + _EPS))
    w0a = W0[:_C] * (g0 * s)[None, :]
    w0b = W0[_C:] * (g0 * s)[None, :]
    w1 = W1 * (g1 * s)[None, :]
    w2 = W2 * (g2 * s)[None, :]
    wsc = Wsc * (gsc * s)[None, :]

    points_t = jnp.swapaxes(points, 1, 2)
    idx = _topk_call(points, points_t)
    feat_flat = features.reshape(_B * _N, _C)
    knn = _gather_call()(feat_flat, idx.reshape(_E // _CHUNK, _CHUNK))
    out = _mlp_call(knn, feat_flat, w0a, w0b, w1, w2, wsc,
                    b0[None, :], b1[None, :], b2[None, :], bsc[None, :])
    return out.reshape(_B, _N, _C)


# speedup vs baseline: 12.6976x; 1.0604x over previous
import functools

import jax
import jax.numpy as jnp
from jax import lax
from jax.experimental import pallas as pl
from jax.experimental.pallas import tpu as pltpu
from jax.experimental.pallas import tpu_sc as plsc

_B, _N, _K, _DP, _C = 8, 2048, 16, 3, 64
_EPS = 1e-3
_RT = 512
_R2 = 512
_E = _B * _N * _K
_NC, _NS = 2, 16
_NW = _NC * _NS
_EPW = _E // _NW
_CHUNK = 128
_NCHUNK = _EPW // _CHUNK


def _topk_body(pts_ref, ptsT_ref, idx_ref):
    b = pl.program_id(0)
    p_tile = pts_ref[0]
    p_all_t = ptsT_ref[0]
    r_tile = jnp.sum(p_tile * p_tile, axis=1, keepdims=True)
    r_all = jnp.sum(p_all_t * p_all_t, axis=0, keepdims=True)
    m = jnp.dot(p_tile, p_all_t, preferred_element_type=jnp.float32)
    work = r_tile - 2.0 * m + r_all
    iota = lax.broadcasted_iota(jnp.int32, (_RT, _N), 1)
    cols = []
    for k in range(_K + 1):
        mval = jnp.min(work, axis=1, keepdims=True)
        cand = jnp.where(work == mval, iota, _N)
        midx = jnp.min(cand, axis=1, keepdims=True)
        if k > 0:
            cols.append(midx)
        work = jnp.where(iota == midx, jnp.float32(jnp.inf), work)
    idx_ref[0] = jnp.concatenate(cols, axis=1) + b * _N


_topk_call = pl.pallas_call(
    _topk_body,
    grid=(_B, _N // _RT),
    in_specs=[
        pl.BlockSpec((1, _RT, _DP), lambda b, i: (b, i, 0)),
        pl.BlockSpec((1, _DP, _N), lambda b, i: (b, 0, 0)),
    ],
    out_specs=pl.BlockSpec((1, _RT, _K + 1 - 1), lambda b, i: (b, i, 0)),
    out_shape=jax.ShapeDtypeStruct((_B, _N, _K), jnp.int32),
)


_CP = 128


def _gather_body(feat_hbm, idx_hbm, out_hbm, idx_v, rows_v, sem):
    wid = lax.axis_index("s") * _NC + lax.axis_index("c")
    base = wid * _EPW
    pltpu.sync_copy(idx_hbm.at[pl.ds(wid * _NCHUNK, _NCHUNK)], idx_v)

    def chunk(j, carry):
        pltpu.async_copy(feat_hbm.at[idx_v.at[j]], rows_v, sem).wait()
        pltpu.sync_copy(rows_v, out_hbm.at[pl.ds(base + j * _CHUNK, _CHUNK)])
        return carry

    lax.fori_loop(0, _NCHUNK, chunk, 0)


@functools.lru_cache(maxsize=1)
def _gather_call():
    return pl.kernel(
        _gather_body,
        out_type=jax.ShapeDtypeStruct((_E, _CP), jnp.float32),
        mesh=plsc.VectorSubcoreMesh(core_axis_name="c", subcore_axis_name="s"),
        scratch_types=[
            pltpu.VMEM((_NCHUNK, _CHUNK), jnp.int32),
            pltpu.VMEM((_CHUNK, _CP), jnp.float32),
            pltpu.SemaphoreType.DMA,
        ],
    )


def _mlp_body(knn_ref, f_ref, w0a_ref, w0b_ref, w1_ref, w2_ref, wsc_ref,
              b0_ref, b1_ref, b2_ref, bsc_ref, o_ref):
    f = f_ref[...]
    d = knn_ref[:, :_C].reshape(_R2, _K, _C) - f[:, None, :]
    y0 = jnp.dot(f, w0a_ref[...], preferred_element_type=jnp.float32)
    z0 = jnp.dot(d.reshape(_R2 * _K, _C), w0b_ref[...],
                 preferred_element_type=jnp.float32)
    a0 = jnp.maximum(z0.reshape(_R2, _K, _C) + y0[:, None, :] + b0_ref[...],
                     0.0)
    a1 = jnp.maximum(
        jnp.dot(a0.reshape(_R2 * _K, _C), w1_ref[...],
                preferred_element_type=jnp.float32) + b1_ref[...], 0.0)
    a2 = jnp.maximum(
        jnp.dot(a1, w2_ref[...], preferred_element_type=jnp.float32)
        + b2_ref[...], 0.0)
    fts = jnp.sum(a2.reshape(_R2, _K, _C), axis=1) * (1.0 / _K)
    sc = jnp.dot(f, wsc_ref[...], preferred_element_type=jnp.float32) \
        + bsc_ref[...]
    o_ref[...] = jnp.maximum(sc + fts, 0.0)


_mlp_call = pl.pallas_call(
    _mlp_body,
    grid=(_B * _N // _R2,),
    in_specs=[
        pl.BlockSpec((_R2 * _K, _CP), lambda i: (i, 0)),
        pl.BlockSpec((_R2, _C), lambda i: (i, 0)),
    ] + [pl.BlockSpec((_C, _C), lambda i: (0, 0))] * 5
      + [pl.BlockSpec((1, _C), lambda i: (0, 0))] * 4,
    out_specs=pl.BlockSpec((_R2, _C), lambda i: (i, 0)),
    out_shape=jax.ShapeDtypeStruct((_B * _N, _C), jnp.float32),
)


def kernel(points, features, W0, W1, W2, Wsc,
           g0, b0, g1, b1, g2, b2, gsc, bsc):
    s = 1.0 / jnp.sqrt(jnp.float32(1.0 + _EPS))
    w0a = W0[:_C] * (g0 * s)[None, :]
    w0b = W0[_C:] * (g0 * s)[None, :]
    w1 = W1 * (g1 * s)[None, :]
    w2 = W2 * (g2 * s)[None, :]
    wsc = Wsc * (gsc * s)[None, :]

    points_t = jnp.swapaxes(points, 1, 2)
    idx = _topk_call(points, points_t)
    feat_flat = features.reshape(_B * _N, _C)
    feat_pad = jnp.pad(feat_flat, ((0, 0), (0, _CP - _C)))
    knn = _gather_call()(feat_pad, idx.reshape(_E // _CHUNK, _CHUNK))
    out = _mlp_call(knn, feat_flat, w0a, w0b, w1, w2, wsc,
                    b0[None, :], b1[None, :], b2[None, :], bsc[None, :])
    return out.reshape(_B, _N, _C)


# speedup vs baseline: 16.2390x; 1.2789x over previous
import functools

import jax
import jax.numpy as jnp
from jax import lax
from jax.experimental import pallas as pl
from jax.experimental.pallas import tpu as pltpu
from jax.experimental.pallas import tpu_sc as plsc

_B, _N, _K, _DP, _C = 8, 2048, 16, 3, 64
_EPS = 1e-3
_RT = 512
_R2 = 512
_E = _B * _N * _K
_NC, _NS = 2, 16
_NW = _NC * _NS
_EPW = _E // _NW
_CHUNK = 128
_NCHUNK = _EPW // _CHUNK


def _topk_body(pts_ref, ptsT_ref, idx_ref):
    b = pl.program_id(0)
    p_tile = pts_ref[0]
    p_all_t = ptsT_ref[0]
    r_tile = jnp.sum(p_tile * p_tile, axis=1, keepdims=True)
    r_all = jnp.sum(p_all_t * p_all_t, axis=0, keepdims=True)
    m = jnp.dot(p_tile, p_all_t, preferred_element_type=jnp.float32)
    d = r_tile - 2.0 * m + r_all
    ib = lax.bitcast_convert_type(d + 128.0, jnp.int32)
    ik = jnp.minimum((ib - 0x43000000) >> 2, 0xFFFFF)
    iota = lax.broadcasted_iota(jnp.int32, (_RT, _N), 1)
    key = (ik << 11) | iota
    big = jnp.int32(0x7FFFFFFF)
    cols = []
    kmin = jnp.min(key, axis=1, keepdims=True)
    for _ in range(_K):
        kmin = jnp.min(jnp.where(key > kmin, key, big), axis=1,
                       keepdims=True)
        cols.append(kmin & 0x7FF)
    idx_ref[0] = jnp.concatenate(cols, axis=1) + b * _N


_topk_call = pl.pallas_call(
    _topk_body,
    grid=(_B, _N // _RT),
    in_specs=[
        pl.BlockSpec((1, _RT, _DP), lambda b, i: (b, i, 0)),
        pl.BlockSpec((1, _DP, _N), lambda b, i: (b, 0, 0)),
    ],
    out_specs=pl.BlockSpec((1, _RT, _K + 1 - 1), lambda b, i: (b, i, 0)),
    out_shape=jax.ShapeDtypeStruct((_B, _N, _K), jnp.int32),
)


_CP = 128


def _gather_body(feat_hbm, idx_hbm, out_hbm, idx_v, rows_v, sem):
    wid = lax.axis_index("s") * _NC + lax.axis_index("c")
    base = wid * _EPW
    pltpu.sync_copy(idx_hbm.at[pl.ds(wid * _NCHUNK, _NCHUNK)], idx_v)

    def chunk(j, carry):
        pltpu.async_copy(feat_hbm.at[idx_v.at[j]], rows_v, sem).wait()
        pltpu.sync_copy(rows_v, out_hbm.at[pl.ds(base + j * _CHUNK, _CHUNK)])
        return carry

    lax.fori_loop(0, _NCHUNK, chunk, 0)


@functools.lru_cache(maxsize=1)
def _gather_call():
    return pl.kernel(
        _gather_body,
        out_type=jax.ShapeDtypeStruct((_E, _CP), jnp.float32),
        mesh=plsc.VectorSubcoreMesh(core_axis_name="c", subcore_axis_name="s"),
        scratch_types=[
            pltpu.VMEM((_NCHUNK, _CHUNK), jnp.int32),
            pltpu.VMEM((_CHUNK, _CP), jnp.float32),
            pltpu.SemaphoreType.DMA,
        ],
    )


def _mlp_body(knn_ref, f_ref, w0a_ref, w0b_ref, w1_ref, w2_ref, wsc_ref,
              b0_ref, b1_ref, b2_ref, bsc_ref, o_ref):
    f = f_ref[...]
    d = knn_ref[:, :_C].reshape(_R2, _K, _C) - f[:, None, :]
    y0 = jnp.dot(f, w0a_ref[...], preferred_element_type=jnp.float32)
    z0 = jnp.dot(d.reshape(_R2 * _K, _C), w0b_ref[...],
                 preferred_element_type=jnp.float32)
    a0 = jnp.maximum(z0.reshape(_R2, _K, _C) + y0[:, None, :] + b0_ref[...],
                     0.0)
    a1 = jnp.maximum(
        jnp.dot(a0.reshape(_R2 * _K, _C), w1_ref[...],
                preferred_element_type=jnp.float32) + b1_ref[...], 0.0)
    a2 = jnp.maximum(
        jnp.dot(a1, w2_ref[...], preferred_element_type=jnp.float32)
        + b2_ref[...], 0.0)
    fts = jnp.sum(a2.reshape(_R2, _K, _C), axis=1) * (1.0 / _K)
    sc = jnp.dot(f, wsc_ref[...], preferred_element_type=jnp.float32) \
        + bsc_ref[...]
    o_ref[...] = jnp.maximum(sc + fts, 0.0)


_mlp_call = pl.pallas_call(
    _mlp_body,
    grid=(_B * _N // _R2,),
    in_specs=[
        pl.BlockSpec((_R2 * _K, _CP), lambda i: (i, 0)),
        pl.BlockSpec((_R2, _C), lambda i: (i, 0)),
    ] + [pl.BlockSpec((_C, _C), lambda i: (0, 0))] * 5
      + [pl.BlockSpec((1, _C), lambda i: (0, 0))] * 4,
    out_specs=pl.BlockSpec((_R2, _C), lambda i: (i, 0)),
    out_shape=jax.ShapeDtypeStruct((_B * _N, _C), jnp.float32),
)


def kernel(points, features, W0, W1, W2, Wsc,
           g0, b0, g1, b1, g2, b2, gsc, bsc):
    s = 1.0 / jnp.sqrt(jnp.float32(1.0 + _EPS))
    w0a = W0[:_C] * (g0 * s)[None, :]
    w0b = W0[_C:] * (g0 * s)[None, :]
    w1 = W1 * (g1 * s)[None, :]
    w2 = W2 * (g2 * s)[None, :]
    wsc = Wsc * (gsc * s)[None, :]

    points_t = jnp.swapaxes(points, 1, 2)
    idx = _topk_call(points, points_t)
    feat_flat = features.reshape(_B * _N, _C)
    feat_pad = jnp.pad(feat_flat, ((0, 0), (0, _CP - _C)))
    knn = _gather_call()(feat_pad, idx.reshape(_E // _CHUNK, _CHUNK))
    out = _mlp_call(knn, feat_flat, w0a, w0b, w1, w2, wsc,
                    b0[None, :], b1[None, :], b2[None, :], bsc[None, :])
    return out.reshape(_B, _N, _C)


# speedup vs baseline: 18.6670x; 1.1495x over previous
import functools

import jax
import jax.numpy as jnp
from jax import lax
from jax.experimental import pallas as pl
from jax.experimental.pallas import tpu as pltpu
from jax.experimental.pallas import tpu_sc as plsc

_B, _N, _K, _DP, _C = 8, 2048, 16, 3, 64
_EPS = 1e-3
_RT = 512
_R2 = 512
_E = _B * _N * _K
_NC, _NS = 2, 16
_NW = _NC * _NS
_EPW = _E // _NW
_CHUNK = 128
_NCHUNK = _EPW // _CHUNK
_HALVES = 2


def _topk_body(b_off, pts_ref, ptsT_ref, idx_ref):
    b = pl.program_id(0) + b_off
    p_tile = pts_ref[0]
    p_all_t = ptsT_ref[0]
    r_tile = jnp.sum(p_tile * p_tile, axis=1, keepdims=True)
    r_all = jnp.sum(p_all_t * p_all_t, axis=0, keepdims=True)
    m = jnp.dot(p_tile, p_all_t, preferred_element_type=jnp.float32)
    d = r_tile - 2.0 * m + r_all
    ib = lax.bitcast_convert_type(d + 128.0, jnp.int32)
    ik = jnp.minimum((ib - 0x43000000) >> 2, 0xFFFFF)
    iota = lax.broadcasted_iota(jnp.int32, (_RT, _N), 1)
    key = (ik << 11) | iota
    big = jnp.int32(0x7FFFFFFF)
    cols = []
    kmin = jnp.min(key, axis=1, keepdims=True)
    for _ in range(_K):
        kmin = jnp.min(jnp.where(key > kmin, key, big), axis=1,
                       keepdims=True)
        cols.append(kmin & 0x7FF)
    idx_ref[0] = jnp.concatenate(cols, axis=1) + b * _N


@functools.lru_cache(maxsize=None)
def _topk_call(nb, b_off):
    return pl.pallas_call(
        functools.partial(_topk_body, b_off),
        grid=(nb, _N // _RT),
        in_specs=[
            pl.BlockSpec((1, _RT, _DP), lambda b, i: (b + b_off, i, 0)),
            pl.BlockSpec((1, _DP, _N), lambda b, i: (b + b_off, 0, 0)),
        ],
        out_specs=pl.BlockSpec((1, _RT, _K), lambda b, i: (b, i, 0)),
        out_shape=jax.ShapeDtypeStruct((nb, _N, _K), jnp.int32),
    )


_CP = 128


def _gather_body(nchunk, feat_hbm, idx_hbm, out_hbm, idx_v, rows_v, sem):
    wid = lax.axis_index("s") * _NC + lax.axis_index("c")
    base = wid * nchunk * _CHUNK
    pltpu.sync_copy(idx_hbm.at[pl.ds(wid * nchunk, nchunk)], idx_v)

    def chunk(j, carry):
        pltpu.async_copy(feat_hbm.at[idx_v.at[j]], rows_v, sem).wait()
        pltpu.sync_copy(rows_v, out_hbm.at[pl.ds(base + j * _CHUNK, _CHUNK)])
        return carry

    lax.fori_loop(0, nchunk, chunk, 0)


@functools.lru_cache(maxsize=None)
def _gather_call(ne):
    nchunk = ne // _NW // _CHUNK
    return pl.kernel(
        functools.partial(_gather_body, nchunk),
        out_type=jax.ShapeDtypeStruct((ne, _CP), jnp.float32),
        mesh=plsc.VectorSubcoreMesh(core_axis_name="c", subcore_axis_name="s"),
        scratch_types=[
            pltpu.VMEM((nchunk, _CHUNK), jnp.int32),
            pltpu.VMEM((_CHUNK, _CP), jnp.float32),
            pltpu.SemaphoreType.DMA,
        ],
    )


def _mlp_body(knn_ref, f_ref, w0a_ref, w0b_ref, w1_ref, w2_ref, wsc_ref,
              b0_ref, b1_ref, b2_ref, bsc_ref, o_ref):
    f = f_ref[...]
    d = knn_ref[:, :_C].reshape(_R2, _K, _C) - f[:, None, :]
    y0 = jnp.dot(f, w0a_ref[...], preferred_element_type=jnp.float32)
    z0 = jnp.dot(d.reshape(_R2 * _K, _C), w0b_ref[...],
                 preferred_element_type=jnp.float32)
    a0 = jnp.maximum(z0.reshape(_R2, _K, _C) + y0[:, None, :] + b0_ref[...],
                     0.0)
    a1 = jnp.maximum(
        jnp.dot(a0.reshape(_R2 * _K, _C), w1_ref[...],
                preferred_element_type=jnp.float32) + b1_ref[...], 0.0)
    a2 = jnp.maximum(
        jnp.dot(a1, w2_ref[...], preferred_element_type=jnp.float32)
        + b2_ref[...], 0.0)
    fts = jnp.sum(a2.reshape(_R2, _K, _C), axis=1) * (1.0 / _K)
    sc = jnp.dot(f, wsc_ref[...], preferred_element_type=jnp.float32) \
        + bsc_ref[...]
    o_ref[...] = jnp.maximum(sc + fts, 0.0)


@functools.lru_cache(maxsize=None)
def _mlp_call(nrows, row_off):
    return pl.pallas_call(
        _mlp_body,
        grid=(nrows // _R2,),
        in_specs=[
            pl.BlockSpec((_R2 * _K, _CP), lambda i: (i, 0)),
            pl.BlockSpec((_R2, _C), lambda i: (i + row_off // _R2, 0)),
        ] + [pl.BlockSpec((_C, _C), lambda i: (0, 0))] * 5
          + [pl.BlockSpec((1, _C), lambda i: (0, 0))] * 4,
        out_specs=pl.BlockSpec((_R2, _C), lambda i: (i, 0)),
        out_shape=jax.ShapeDtypeStruct((nrows, _C), jnp.float32),
    )


def kernel(points, features, W0, W1, W2, Wsc,
           g0, b0, g1, b1, g2, b2, gsc, bsc):
    s = 1.0 / jnp.sqrt(jnp.float32(1.0 + _EPS))
    w0a = W0[:_C] * (g0 * s)[None, :]
    w0b = W0[_C:] * (g0 * s)[None, :]
    w1 = W1 * (g1 * s)[None, :]
    w2 = W2 * (g2 * s)[None, :]
    wsc = Wsc * (gsc * s)[None, :]

    points_t = jnp.swapaxes(points, 1, 2)
    feat_flat = features.reshape(_B * _N, _C)
    feat_pad = jnp.pad(feat_flat, ((0, 0), (0, _CP - _C)))

    nh = _B // _HALVES
    idxs = [_topk_call(nh, h * nh)(points, points_t) for h in range(_HALVES)]
    outs = []
    for h in range(_HALVES):
        ne = nh * _N * _K
        knn = _gather_call(ne)(feat_pad, idxs[h].reshape(ne // _CHUNK, _CHUNK))
        outs.append(_mlp_call(nh * _N, h * nh * _N)(
            knn, feat_flat, w0a, w0b, w1, w2, wsc,
            b0[None, :], b1[None, :], b2[None, :], bsc[None, :]))
    return jnp.concatenate(outs).reshape(_B, _N, _C)


# speedup vs baseline: 19.0776x; 1.0220x over previous
import functools

import jax
import jax.numpy as jnp
from jax import lax
from jax.experimental import pallas as pl
from jax.experimental.pallas import tpu as pltpu
from jax.experimental.pallas import tpu_sc as plsc

_B, _N, _K, _DP, _C = 8, 2048, 16, 3, 64
_EPS = 1e-3
_RT = 512
_R2 = 512
_E = _B * _N * _K
_NC, _NS = 2, 16
_NW = _NC * _NS
_EPW = _E // _NW
_CHUNK = 128
_NCHUNK = _EPW // _CHUNK
_HALVES = 4


def _topk_body(b_off, pts_ref, ptsT_ref, idx_ref):
    b = pl.program_id(0) + b_off
    p_tile = pts_ref[0]
    p_all_t = ptsT_ref[0]
    r_tile = jnp.sum(p_tile * p_tile, axis=1, keepdims=True)
    r_all = jnp.sum(p_all_t * p_all_t, axis=0, keepdims=True)
    m = jnp.dot(p_tile, p_all_t, preferred_element_type=jnp.float32)
    d = r_tile - 2.0 * m + r_all
    ib = lax.bitcast_convert_type(d + 128.0, jnp.int32)
    ik = jnp.minimum((ib - 0x43000000) >> 2, 0xFFFFF)
    iota = lax.broadcasted_iota(jnp.int32, (_RT, _N), 1)
    key = (ik << 11) | iota
    big = jnp.int32(0x7FFFFFFF)
    cols = []
    kmin = jnp.min(key, axis=1, keepdims=True)
    for _ in range(_K):
        kmin = jnp.min(jnp.where(key > kmin, key, big), axis=1,
                       keepdims=True)
        cols.append(kmin & 0x7FF)
    idx_ref[0] = jnp.concatenate(cols, axis=1) + b * _N


@functools.lru_cache(maxsize=None)
def _topk_call(nb, b_off):
    return pl.pallas_call(
        functools.partial(_topk_body, b_off),
        grid=(nb, _N // _RT),
        in_specs=[
            pl.BlockSpec((1, _RT, _DP), lambda b, i: (b + b_off, i, 0)),
            pl.BlockSpec((1, _DP, _N), lambda b, i: (b + b_off, 0, 0)),
        ],
        out_specs=pl.BlockSpec((1, _RT, _K), lambda b, i: (b, i, 0)),
        out_shape=jax.ShapeDtypeStruct((nb, _N, _K), jnp.int32),
    )


_CP = 128


def _gather_body(nchunk, feat_hbm, idx_hbm, out_hbm, idx_v, rows_v, sem):
    wid = lax.axis_index("s") * _NC + lax.axis_index("c")
    base = wid * nchunk * _CHUNK
    pltpu.sync_copy(idx_hbm.at[pl.ds(wid * nchunk, nchunk)], idx_v)

    def chunk(j, carry):
        pltpu.async_copy(feat_hbm.at[idx_v.at[j]], rows_v, sem).wait()
        pltpu.sync_copy(rows_v, out_hbm.at[pl.ds(base + j * _CHUNK, _CHUNK)])
        return carry

    lax.fori_loop(0, nchunk, chunk, 0)


@functools.lru_cache(maxsize=None)
def _gather_call(ne):
    nchunk = ne // _NW // _CHUNK
    return pl.kernel(
        functools.partial(_gather_body, nchunk),
        out_type=jax.ShapeDtypeStruct((ne, _CP), jnp.float32),
        mesh=plsc.VectorSubcoreMesh(core_axis_name="c", subcore_axis_name="s"),
        scratch_types=[
            pltpu.VMEM((nchunk, _CHUNK), jnp.int32),
            pltpu.VMEM((_CHUNK, _CP), jnp.float32),
            pltpu.SemaphoreType.DMA,
        ],
    )


def _mlp_body(knn_ref, f_ref, w0a_ref, w0b_ref, w1_ref, w2_ref, wsc_ref,
              b0_ref, b1_ref, b2_ref, bsc_ref, o_ref):
    f = f_ref[...]
    d = knn_ref[:, :_C].reshape(_R2, _K, _C) - f[:, None, :]
    y0 = jnp.dot(f, w0a_ref[...], preferred_element_type=jnp.float32)
    z0 = jnp.dot(d.reshape(_R2 * _K, _C), w0b_ref[...],
                 preferred_element_type=jnp.float32)
    a0 = jnp.maximum(z0.reshape(_R2, _K, _C) + y0[:, None, :] + b0_ref[...],
                     0.0)
    a1 = jnp.maximum(
        jnp.dot(a0.reshape(_R2 * _K, _C), w1_ref[...],
                preferred_element_type=jnp.float32) + b1_ref[...], 0.0)
    a2 = jnp.maximum(
        jnp.dot(a1, w2_ref[...], preferred_element_type=jnp.float32)
        + b2_ref[...], 0.0)
    fts = jnp.sum(a2.reshape(_R2, _K, _C), axis=1) * (1.0 / _K)
    sc = jnp.dot(f, wsc_ref[...], preferred_element_type=jnp.float32) \
        + bsc_ref[...]
    o_ref[...] = jnp.maximum(sc + fts, 0.0)


@functools.lru_cache(maxsize=None)
def _mlp_call(nrows, row_off):
    return pl.pallas_call(
        _mlp_body,
        grid=(nrows // _R2,),
        in_specs=[
            pl.BlockSpec((_R2 * _K, _CP), lambda i: (i, 0)),
            pl.BlockSpec((_R2, _C), lambda i: (i + row_off // _R2, 0)),
        ] + [pl.BlockSpec((_C, _C), lambda i: (0, 0))] * 5
          + [pl.BlockSpec((1, _C), lambda i: (0, 0))] * 4,
        out_specs=pl.BlockSpec((_R2, _C), lambda i: (i, 0)),
        out_shape=jax.ShapeDtypeStruct((nrows, _C), jnp.float32),
    )


def kernel(points, features, W0, W1, W2, Wsc,
           g0, b0, g1, b1, g2, b2, gsc, bsc):
    s = 1.0 / jnp.sqrt(jnp.float32(1.0 + _EPS))
    w0a = W0[:_C] * (g0 * s)[None, :]
    w0b = W0[_C:] * (g0 * s)[None, :]
    w1 = W1 * (g1 * s)[None, :]
    w2 = W2 * (g2 * s)[None, :]
    wsc = Wsc * (gsc * s)[None, :]

    points_t = jnp.swapaxes(points, 1, 2)
    feat_flat = features.reshape(_B * _N, _C)
    feat_pad = jnp.pad(feat_flat, ((0, 0), (0, _CP - _C)))

    nh = _B // _HALVES
    idxs = [_topk_call(nh, h * nh)(points, points_t) for h in range(_HALVES)]
    outs = []
    for h in range(_HALVES):
        ne = nh * _N * _K
        knn = _gather_call(ne)(feat_pad, idxs[h].reshape(ne // _CHUNK, _CHUNK))
        outs.append(_mlp_call(nh * _N, h * nh * _N)(
            knn, feat_flat, w0a, w0b, w1, w2, wsc,
            b0[None, :], b1[None, :], b2[None, :], bsc[None, :]))
    return jnp.concatenate(outs).reshape(_B, _N, _C)


# speedup vs baseline: 23.2566x; 1.2191x over previous
import functools

import jax
import jax.numpy as jnp
from jax import lax
from jax.experimental import pallas as pl
from jax.experimental.pallas import tpu as pltpu
from jax.experimental.pallas import tpu_sc as plsc

_B, _N, _K, _DP, _C = 8, 2048, 16, 3, 64
_EPS = 1e-3
_RT = 512
_R2 = 512
_E = _B * _N * _K
_NC, _NS = 2, 16
_NW = _NC * _NS
_EPW = _E // _NW
_CHUNK = 128
_NCHUNK = _EPW // _CHUNK
_HALVES = 4


def _topk_body(b_off, pts_ref, ptsT_ref, idx_ref):
    b = pl.program_id(0) + b_off
    p_tile = pts_ref[0]
    p_all_t = ptsT_ref[0]
    r_tile = jnp.sum(p_tile * p_tile, axis=1, keepdims=True)
    r_all = jnp.sum(p_all_t * p_all_t, axis=0, keepdims=True)
    m = jnp.dot(p_tile, p_all_t, preferred_element_type=jnp.float32)
    d = r_tile - 2.0 * m + r_all
    ib = lax.bitcast_convert_type(d + 128.0, jnp.int32)
    ik = jnp.clip((ib - 0x43000000) >> 2, -0x800, 0xF0000)
    iota = lax.broadcasted_iota(jnp.int32, (_RT, _N), 1)
    fkey = lax.bitcast_convert_type(((ik + 0x1000) << 11) | iota,
                                    jnp.float32)
    big = jnp.float32(1e38)
    cols = []
    kmin = jnp.min(fkey, axis=1, keepdims=True)
    for _ in range(_K):
        kmin = jnp.min(jnp.where(fkey > kmin, fkey, big), axis=1,
                       keepdims=True)
        cols.append(lax.bitcast_convert_type(kmin, jnp.int32) & 0x7FF)
    idx_ref[0] = jnp.concatenate(cols, axis=1) + b * _N


@functools.lru_cache(maxsize=None)
def _topk_call(nb, b_off):
    return pl.pallas_call(
        functools.partial(_topk_body, b_off),
        grid=(nb, _N // _RT),
        in_specs=[
            pl.BlockSpec((1, _RT, _DP), lambda b, i: (b + b_off, i, 0)),
            pl.BlockSpec((1, _DP, _N), lambda b, i: (b + b_off, 0, 0)),
        ],
        out_specs=pl.BlockSpec((1, _RT, _K), lambda b, i: (b, i, 0)),
        out_shape=jax.ShapeDtypeStruct((nb, _N, _K), jnp.int32),
    )


_CP = 128


def _gather_body(nchunk, feat_hbm, idx_hbm, out_hbm, idx_v, rows_v, sem):
    wid = lax.axis_index("s") * _NC + lax.axis_index("c")
    base = wid * nchunk * _CHUNK
    pltpu.sync_copy(idx_hbm.at[pl.ds(wid * nchunk, nchunk)], idx_v)

    def chunk(j, carry):
        pltpu.async_copy(feat_hbm.at[idx_v.at[j]], rows_v, sem).wait()
        pltpu.sync_copy(rows_v, out_hbm.at[pl.ds(base + j * _CHUNK, _CHUNK)])
        return carry

    lax.fori_loop(0, nchunk, chunk, 0)


@functools.lru_cache(maxsize=None)
def _gather_call(ne):
    nchunk = ne // _NW // _CHUNK
    return pl.kernel(
        functools.partial(_gather_body, nchunk),
        out_type=jax.ShapeDtypeStruct((ne, _CP), jnp.float32),
        mesh=plsc.VectorSubcoreMesh(core_axis_name="c", subcore_axis_name="s"),
        scratch_types=[
            pltpu.VMEM((nchunk, _CHUNK), jnp.int32),
            pltpu.VMEM((_CHUNK, _CP), jnp.float32),
            pltpu.SemaphoreType.DMA,
        ],
    )


def _mlp_body(knn_ref, f_ref, w0a_ref, w0b_ref, w1_ref, w2_ref, wsc_ref,
              b0_ref, b1_ref, b2_ref, bsc_ref, o_ref):
    f = f_ref[...]
    d = knn_ref[:, :_C].reshape(_R2, _K, _C) - f[:, None, :]
    y0 = jnp.dot(f, w0a_ref[...], preferred_element_type=jnp.float32)
    z0 = jnp.dot(d.reshape(_R2 * _K, _C), w0b_ref[...],
                 preferred_element_type=jnp.float32)
    a0 = jnp.maximum(z0.reshape(_R2, _K, _C) + y0[:, None, :] + b0_ref[...],
                     0.0)
    a1 = jnp.maximum(
        jnp.dot(a0.reshape(_R2 * _K, _C), w1_ref[...],
                preferred_element_type=jnp.float32) + b1_ref[...], 0.0)
    a2 = jnp.maximum(
        jnp.dot(a1, w2_ref[...], preferred_element_type=jnp.float32)
        + b2_ref[...], 0.0)
    fts = jnp.sum(a2.reshape(_R2, _K, _C), axis=1) * (1.0 / _K)
    sc = jnp.dot(f, wsc_ref[...], preferred_element_type=jnp.float32) \
        + bsc_ref[...]
    o_ref[...] = jnp.maximum(sc + fts, 0.0)


@functools.lru_cache(maxsize=None)
def _mlp_call(nrows, row_off):
    return pl.pallas_call(
        _mlp_body,
        grid=(nrows // _R2,),
        in_specs=[
            pl.BlockSpec((_R2 * _K, _CP), lambda i: (i, 0)),
            pl.BlockSpec((_R2, _C), lambda i: (i + row_off // _R2, 0)),
        ] + [pl.BlockSpec((_C, _C), lambda i: (0, 0))] * 5
          + [pl.BlockSpec((1, _C), lambda i: (0, 0))] * 4,
        out_specs=pl.BlockSpec((_R2, _C), lambda i: (i, 0)),
        out_shape=jax.ShapeDtypeStruct((nrows, _C), jnp.float32),
    )


def kernel(points, features, W0, W1, W2, Wsc,
           g0, b0, g1, b1, g2, b2, gsc, bsc):
    s = 1.0 / jnp.sqrt(jnp.float32(1.0 + _EPS))
    w0a = W0[:_C] * (g0 * s)[None, :]
    w0b = W0[_C:] * (g0 * s)[None, :]
    w1 = W1 * (g1 * s)[None, :]
    w2 = W2 * (g2 * s)[None, :]
    wsc = Wsc * (gsc * s)[None, :]

    points_t = jnp.swapaxes(points, 1, 2)
    feat_flat = features.reshape(_B * _N, _C)
    feat_pad = jnp.pad(feat_flat, ((0, 0), (0, _CP - _C)))

    nh = _B // _HALVES
    idxs = [_topk_call(nh, h * nh)(points, points_t) for h in range(_HALVES)]
    outs = []
    for h in range(_HALVES):
        ne = nh * _N * _K
        knn = _gather_call(ne)(feat_pad, idxs[h].reshape(ne // _CHUNK, _CHUNK))
        outs.append(_mlp_call(nh * _N, h * nh * _N)(
            knn, feat_flat, w0a, w0b, w1, w2, wsc,
            b0[None, :], b1[None, :], b2[None, :], bsc[None, :]))
    return jnp.concatenate(outs).reshape(_B, _N, _C)
